# baseline (speedup 1.0000x reference)
import functools

import jax
import jax.numpy as jnp
from jax import lax
from jax.experimental import pallas as pl
from jax.experimental.pallas import tpu as pltpu
from jax.experimental.pallas import tpu_sc as plsc

_VOCAB = 100000
_EMBED = 128
_B = 4096
_L = 50

_NC = 2
_NS = 16
_NW = _NC * _NS

_B_PER_W = _B // _NW
_LPC = 2
_NCHUNK = _L // _LPC
_NBUF = 3


@functools.partial(
    pl.kernel,
    out_type=jax.ShapeDtypeStruct((_L, _B, _EMBED), jnp.float32),
    mesh=plsc.VectorSubcoreMesh(core_axis_name="c", subcore_axis_name="s"),
    scratch_types=[
        pltpu.VMEM((_L, _B_PER_W), jnp.int32),
        pltpu.VMEM((_LPC, _B_PER_W, _EMBED), jnp.float32),
        pltpu.VMEM((_LPC, _B_PER_W, _EMBED), jnp.float32),
        pltpu.VMEM((_LPC, _B_PER_W, _EMBED), jnp.float32),
        pltpu.SemaphoreType.DMA,
        pltpu.SemaphoreType.DMA,
    ],
)
def _gather_kernel(table_hbm, idx_hbm, out_hbm, idx_v, buf0, buf1, buf2, sem_g, sem_w):
    wid = lax.axis_index("s") * _NC + lax.axis_index("c")
    b0 = wid * _B_PER_W
    pltpu.sync_copy(idx_hbm.at[:, pl.ds(b0, _B_PER_W)], idx_v)

    bufs = (buf0, buf1, buf2)

    def fire_gather(c):
        return [
            pltpu.async_copy(
                table_hbm.at[idx_v.at[c * _LPC + k]],
                bufs[c % _NBUF].at[k],
                sem_g,
            )
            for k in range(_LPC)
        ]

    def fire_write(c):
        return pltpu.async_copy(
            bufs[c % _NBUF],
            out_hbm.at[pl.ds(c * _LPC, _LPC), pl.ds(b0, _B_PER_W)],
            sem_w,
        )

    gathers = [fire_gather(0), fire_gather(1)]
    writes = []
    for c in range(_NCHUNK):
        if c >= 2:
            writes[c - 2].wait()
        if c + 2 < _NCHUNK:
            gathers.append(fire_gather(c + 2))
        for g in gathers[c]:
            g.wait()
        writes.append(fire_write(c))
    writes[_NCHUNK - 2].wait()
    writes[_NCHUNK - 1].wait()


def kernel(inputs, w, b):
    idx_t = jnp.transpose(inputs).astype(jnp.int32)
    out = _gather_kernel(w, idx_t)
    return jnp.transpose(out, (1, 0, 2))


# speedup vs baseline: 1.0129x; 1.0129x over previous
import functools

import jax
import jax.numpy as jnp
from jax import lax
from jax.experimental import pallas as pl
from jax.experimental.pallas import tpu as pltpu
from jax.experimental.pallas import tpu_sc as plsc

_VOCAB = 100000
_EMBED = 128
_B = 4096
_L = 50

_NC = 2
_NS = 16

_LG = 2
_BG = 16
_L_PER_W = _L // _LG
_B_PER_W = _B // _BG
_PER_W = _L_PER_W * _B_PER_W
_NBUF = 3


@functools.partial(
    pl.kernel,
    out_type=jax.ShapeDtypeStruct((_L, _B, _EMBED), jnp.float32),
    mesh=plsc.VectorSubcoreMesh(core_axis_name="c", subcore_axis_name="s"),
    scratch_types=[
        pltpu.VMEM((_PER_W,), jnp.int32),
        pltpu.VMEM((_B_PER_W, _EMBED), jnp.float32),
        pltpu.VMEM((_B_PER_W, _EMBED), jnp.float32),
        pltpu.VMEM((_B_PER_W, _EMBED), jnp.float32),
        pltpu.SemaphoreType.DMA,
        pltpu.SemaphoreType.DMA,
    ],
)
def _gather_kernel(table_hbm, idx_hbm, out_hbm, idx_v, buf0, buf1, buf2, sem_g, sem_w):
    lg = lax.axis_index("c")
    bg = lax.axis_index("s")
    wid = lg * _BG + bg
    l0 = lg * _L_PER_W
    b0 = bg * _B_PER_W
    pltpu.sync_copy(idx_hbm.at[pl.ds(wid * _PER_W, _PER_W)], idx_v)

    bufs = (buf0, buf1, buf2)

    def fire_gather(c):
        return pltpu.async_copy(
            table_hbm.at[idx_v.at[pl.ds(c * _B_PER_W, _B_PER_W)]],
            bufs[c % _NBUF],
            sem_g,
        )

    def fire_write(c):
        return pltpu.async_copy(
            bufs[c % _NBUF], out_hbm.at[l0 + c, pl.ds(b0, _B_PER_W)], sem_w
        )

    gathers = [fire_gather(0), fire_gather(1)]
    writes = []
    for c in range(_L_PER_W):
        if c >= 2:
            writes[c - 2].wait()
        if c + 2 < _L_PER_W:
            gathers.append(fire_gather(c + 2))
        gathers[c].wait()
        writes.append(fire_write(c))
    writes[_L_PER_W - 2].wait()
    writes[_L_PER_W - 1].wait()


def kernel(inputs, w, b):
    idx_t = jnp.transpose(inputs).astype(jnp.int32)
    idx_w = jnp.transpose(
        jnp.reshape(idx_t, (_LG, _L_PER_W, _BG, _B_PER_W)), (0, 2, 1, 3)
    )
    out = _gather_kernel(w, jnp.reshape(idx_w, (_L * _B,)))
    return jnp.transpose(out, (1, 0, 2))
